# baseline (speedup 1.0000x reference)
import numpy as np
import jax
import jax.numpy as jnp
from jax import lax
from jax.experimental import pallas as pl
from jax.experimental.pallas import tpu as pltpu
from jax.experimental.pallas import tpu_sc as plsc

B, N, H = 64, 128, 32
NP1 = N + 1
L = 16
CPL = N // L
CPR = CPL + 1
W = CPR * L
NTCH = NP1 * CPR
NC, NS = 2, 16
NW = NC * NS
GPW = B // NW
NSTAGE = N * N + 128
NI = 4
NGRP = NP1 // NI
TAIL = NP1 - NI * NGRP
NPAIR = (NGRP - 2) // 2
HCH = H // L

_MARKER = np.zeros((8, 128), dtype=np.int32)
_MARKER[0, 0] = 2


def _sc_body(adj_hbm, wv_hbm, vt_hbm, marker_hbm, out_hbm,
             staging, tmpl, buf0, buf1, wv_v, vt_v, sem0, sem1):
    wid = lax.axis_index("s") * NC + lax.axis_index("c")

    pltpu.sync_copy(wv_hbm, wv_v)
    pltpu.sync_copy(vt_hbm, vt_v)
    pltpu.sync_copy(marker_hbm, staging.at[N * N // 1024])

    lanes = lax.iota(jnp.int32, L)
    colN = jnp.full((L,), N, dtype=jnp.int32)

    def fill_slab(i0, ni, buf):
        @plsc.parallel_loop(0, ni * H, step=1)
        def fill(k, i0=i0, buf=buf):
            r = k // H
            h = k - r * H
            hv = jnp.full((L,), h, dtype=jnp.int32)
            wv = plsc.load_gather(wv_v, [hv])
            vt = plsc.load_gather(vt_v, [hv])
            base = (i0 + r) * W
            for c in range(CPL):
                t = tmpl[pl.ds(base + c * L, L)]
                buf[r, h, pl.ds(c * L, L)] = jnp.where(t > 1.5, vt, t * wv)

        @plsc.parallel_loop(0, ni * HCH, step=1)
        def fill_col(k, i0=i0, buf=buf):
            r = k // HCH
            hc = k - r * HCH
            tv = plsc.load_gather(
                tmpl, [jnp.full((L,), (i0 + r) * W + N, dtype=jnp.int32)])
            wvc = wv_v[pl.ds(hc * L, L)]
            vtc = vt_v[pl.ds(hc * L, L)]
            val = jnp.where(tv > 1.5, vtc, tv * wvc)
            plsc.store_scatter(buf.at[r], [lanes + hc * L, colN], val)

    def start(buf, b, i0, ni, sem):
        return pltpu.async_copy(
            buf.at[pl.ds(0, ni)], out_hbm.at[b, pl.ds(i0, ni)], sem)

    def wait(buf, ni, sem):
        pltpu.make_async_copy(
            buf.at[pl.ds(0, ni)], out_hbm.at[0, pl.ds(0, ni)], sem).wait()

    bufs = (buf0, buf1)
    sems = (sem0, sem1)
    pend = [[], []]

    for g in range(GPW):
        b = wid * GPW + g
        pltpu.sync_copy(adj_hbm.at[b], staging.at[pl.ds(0, N * N // 1024)])

        @plsc.parallel_loop(0, NTCH, step=1, unroll=8)
        def build(k):
            i = k // CPR
            c = k - i * CPR
            col = lanes + c * L
            ai = i - 1
            aj = col - 1
            src = (ai // 8) * 1024 + (ai % 8) * N + aj
            idx = jnp.where((col == 0) | (i == 0), N * N,
                            jnp.where(col > N, N * N + 1, src))
            t = plsc.load_gather(
                staging, [idx // 1024, (idx // N) % 8, idx % N])
            tmpl[pl.ds(k * L, L)] = t.astype(jnp.float32)

        for q in (0, 1):
            if pend[q]:
                wait(bufs[q], pend[q].pop(), sems[q])
            fill_slab(q * NI, NI, bufs[q])
            start(bufs[q], b, q * NI, NI, sems[q])

        def body(j, carry):
            i0 = (2 * j + 2) * NI
            wait(buf0, NI, sem0)
            fill_slab(i0, NI, buf0)
            start(buf0, b, i0, NI, sem0)
            wait(buf1, NI, sem1)
            fill_slab(i0 + NI, NI, buf1)
            start(buf1, b, i0 + NI, NI, sem1)
            return carry

        lax.fori_loop(0, NPAIR, body, 0)

        wait(buf0, NI, sem0)
        fill_slab(NGRP * NI, TAIL, buf0)
        start(buf0, b, NGRP * NI, TAIL, sem0)
        pend[0] = [TAIL]
        pend[1] = [NI]

    wait(buf0, pend[0].pop(), sem0)
    wait(buf1, pend[1].pop(), sem1)


def kernel(adj, adj_bias_w, vt_bias_w):
    adj2 = adj.reshape(B, N // 8, 8, N)
    run = pl.kernel(
        _sc_body,
        out_type=jax.ShapeDtypeStruct((B, NP1, H, NP1), jnp.float32),
        mesh=plsc.VectorSubcoreMesh(core_axis_name="c", subcore_axis_name="s"),
        compiler_params=pltpu.CompilerParams(
            needs_layout_passes=False, use_tc_tiling_on_sc=True),
        scratch_types=[
            pltpu.VMEM((N * N // 1024 + 1, 8, 128), jnp.int32),
            pltpu.VMEM((NP1 * W,), jnp.float32),
            pltpu.VMEM((NI, H, NP1), jnp.float32),
            pltpu.VMEM((NI, H, NP1), jnp.float32),
            pltpu.VMEM((128,), jnp.float32),
            pltpu.VMEM((128,), jnp.float32),
            pltpu.SemaphoreType.DMA,
            pltpu.SemaphoreType.DMA,
        ],
    )
    wv = jnp.zeros((128,), jnp.float32).at[:H].set(adj_bias_w[1])
    vt = jnp.zeros((128,), jnp.float32).at[:H].set(vt_bias_w[0])
    out = run(adj2, wv, vt, jnp.asarray(_MARKER))
    return out.transpose(0, 2, 1, 3)


# speedup vs baseline: 1.0916x; 1.0916x over previous
import numpy as np
import jax
import jax.numpy as jnp
from jax import lax
from jax.experimental import pallas as pl
from jax.experimental.pallas import tpu as pltpu
from jax.experimental.pallas import tpu_sc as plsc

B, N, H = 64, 128, 32
NP1 = N + 1
L = 16
CPL = N // L
CPR = CPL + 1
W = CPR * L
NTCH = NP1 * CPR
NC, NS = 2, 16
NW = NC * NS
GPW = B // NW
NSTAGE = N * N + 128
NI = 4
NGRP = NP1 // NI
TAIL = NP1 - NI * NGRP
NPAIR = (NGRP - 2) // 2
HCH = H // L

_MARKER = np.zeros((128,), dtype=np.int32)
_MARKER[0] = 2


def _sc_body(adj_hbm, wv_hbm, vt_hbm, marker_hbm, out_hbm,
             staging, tmpl, buf0, buf1, wv_v, vt_v, sem0, sem1):
    wid = lax.axis_index("s") * NC + lax.axis_index("c")

    pltpu.sync_copy(wv_hbm, wv_v)
    pltpu.sync_copy(vt_hbm, vt_v)
    pltpu.sync_copy(marker_hbm, staging.at[pl.ds(N * N, 128)])

    lanes = lax.iota(jnp.int32, L)
    colN = jnp.full((L,), N, dtype=jnp.int32)

    def fill_slab(i0, ni, buf):
        @plsc.parallel_loop(0, ni * H, step=1, unroll=2)
        def fill(k, i0=i0, buf=buf):
            r = k // H
            h = k - r * H
            hv = jnp.full((L,), h, dtype=jnp.int32)
            wv = plsc.load_gather(wv_v, [hv])
            vt = plsc.load_gather(vt_v, [hv])
            base = (i0 + r) * W
            for c in range(CPL):
                t = tmpl[pl.ds(base + c * L, L)]
                buf[r, h, pl.ds(c * L, L)] = jnp.where(t > 1.5, vt, t * wv)

        @plsc.parallel_loop(0, ni * HCH, step=1)
        def fill_col(k, i0=i0, buf=buf):
            r = k // HCH
            hc = k - r * HCH
            tv = plsc.load_gather(
                tmpl, [jnp.full((L,), (i0 + r) * W + N, dtype=jnp.int32)])
            wvc = wv_v[pl.ds(hc * L, L)]
            vtc = vt_v[pl.ds(hc * L, L)]
            val = jnp.where(tv > 1.5, vtc, tv * wvc)
            plsc.store_scatter(buf.at[r], [lanes + hc * L, colN], val)

    def start(buf, b, i0, ni, sem):
        return pltpu.async_copy(
            buf.at[pl.ds(0, ni)], out_hbm.at[b, pl.ds(i0, ni)], sem)

    def wait(buf, ni, sem):
        pltpu.make_async_copy(
            buf.at[pl.ds(0, ni)], out_hbm.at[0, pl.ds(0, ni)], sem).wait()

    bufs = (buf0, buf1)
    sems = (sem0, sem1)
    pend = [[], []]

    for g in range(GPW):
        b = wid * GPW + g
        pltpu.sync_copy(adj_hbm.at[b], staging.at[pl.ds(0, N * N)])

        @plsc.parallel_loop(0, NTCH, step=1, unroll=8)
        def build(k):
            i = k // CPR
            c = k - i * CPR
            col = lanes + c * L
            src = (i - 1) * N + col - 1
            idx = jnp.where((col == 0) | (i == 0), N * N,
                            jnp.where(col > N, N * N + 1, src))
            t = plsc.load_gather(staging, [idx])
            tmpl[pl.ds(k * L, L)] = t.astype(jnp.float32)

        for q in (0, 1):
            if pend[q]:
                wait(bufs[q], pend[q].pop(), sems[q])
            fill_slab(q * NI, NI, bufs[q])
            start(bufs[q], b, q * NI, NI, sems[q])

        def body(j, carry):
            i0 = (2 * j + 2) * NI
            wait(buf0, NI, sem0)
            fill_slab(i0, NI, buf0)
            start(buf0, b, i0, NI, sem0)
            wait(buf1, NI, sem1)
            fill_slab(i0 + NI, NI, buf1)
            start(buf1, b, i0 + NI, NI, sem1)
            return carry

        lax.fori_loop(0, NPAIR, body, 0)

        wait(buf0, NI, sem0)
        fill_slab(NGRP * NI, TAIL, buf0)
        start(buf0, b, NGRP * NI, TAIL, sem0)
        pend[0] = [TAIL]
        pend[1] = [NI]

    wait(buf0, pend[0].pop(), sem0)
    wait(buf1, pend[1].pop(), sem1)


def kernel(adj, adj_bias_w, vt_bias_w):
    adj2 = adj.reshape(B, N * N)
    run = pl.kernel(
        _sc_body,
        out_type=jax.ShapeDtypeStruct((B, NP1, H, NP1), jnp.float32),
        mesh=plsc.VectorSubcoreMesh(core_axis_name="c", subcore_axis_name="s"),
        compiler_params=pltpu.CompilerParams(
            needs_layout_passes=False, use_tc_tiling_on_sc=True),
        scratch_types=[
            pltpu.VMEM((NSTAGE,), jnp.int32),
            pltpu.VMEM((NP1 * W,), jnp.float32),
            pltpu.VMEM((NI, H, NP1), jnp.float32),
            pltpu.VMEM((NI, H, NP1), jnp.float32),
            pltpu.VMEM((128,), jnp.float32),
            pltpu.VMEM((128,), jnp.float32),
            pltpu.SemaphoreType.DMA,
            pltpu.SemaphoreType.DMA,
        ],
    )
    wv = jnp.zeros((128,), jnp.float32).at[:H].set(adj_bias_w[1])
    vt = jnp.zeros((128,), jnp.float32).at[:H].set(vt_bias_w[0])
    out = run(adj2, wv, vt, jnp.asarray(_MARKER))
    return out.transpose(0, 2, 1, 3)
